# Initial kernel scaffold; baseline (speedup 1.0000x reference)
import jax
import jax.numpy as jnp
from jax.experimental import pallas as pl


def kernel(input_ids, token_type_ids, word_embeddings, position_embeddings, token_type_embeddings, ln_gamma, ln_beta):
    raise NotImplementedError("write your pallas kernel here")


# speedup vs baseline: 1.9514x; 1.9514x over previous
import functools

import jax
import jax.numpy as jnp
from jax import lax
from jax.experimental import pallas as pl
from jax.experimental.pallas import tpu as pltpu
from jax.experimental.pallas import tpu_sc as plsc

B, S, D = 4, 2048, 768
N = B * S
EPS = 1e-12

_info = plsc.get_sparse_core_info()
NC, NS = _info.num_cores, _info.num_subcores
NW = NC * NS
TOK_PER_W = N // NW
CHUNK = 64
NCHUNK = TOK_PER_W // CHUNK


def _sc_gather(word_hbm, ids_hbm, out_hbm, idx_v, rows_v, sem):
    wid = lax.axis_index("s") * NC + lax.axis_index("c")
    pltpu.sync_copy(ids_hbm.at[wid], idx_v)
    base = wid * TOK_PER_W
    for j in range(NCHUNK):
        pltpu.async_copy(word_hbm.at[idx_v.at[j]], rows_v, sem).wait()
        pltpu.sync_copy(rows_v, out_hbm.at[pl.ds(base + j * CHUNK, CHUNK)])


@jax.jit
def _gather_rows(word_embeddings, ids):
    ids3 = ids.reshape(NW, NCHUNK, CHUNK)
    mesh = plsc.VectorSubcoreMesh(core_axis_name="c", subcore_axis_name="s")
    return pl.kernel(
        _sc_gather,
        mesh=mesh,
        out_type=jax.ShapeDtypeStruct((N, D), jnp.float32),
        scratch_types=[
            pltpu.VMEM((NCHUNK, CHUNK), jnp.int32),
            pltpu.VMEM((CHUNK, D), jnp.float32),
            pltpu.SemaphoreType.DMA,
        ],
    )(word_embeddings, ids3)


ROWS_BLK = 256


def _tc_epilogue(g_ref, p_ref, tt_tab_ref, tt_ref, gamma_ref, beta_ref, o_ref):
    x = g_ref[...] + p_ref[...]
    tt = tt_ref[0].astype(jnp.float32)
    row0 = tt_tab_ref[0:1, :]
    row1 = tt_tab_ref[1:2, :]
    x = x + row0 + tt * (row1 - row0)
    mean = jnp.mean(x, axis=-1, keepdims=True)
    d = x - mean
    var = jnp.mean(d * d, axis=-1, keepdims=True)
    o_ref[...] = d * lax.rsqrt(var + EPS) * gamma_ref[...] + beta_ref[...]


@jax.jit
def _epilogue(gathered, position_embeddings, token_type_embeddings, tt_ids,
              ln_gamma, ln_beta):
    nblk = N // ROWS_BLK
    pos_blks = S // ROWS_BLK
    tt3 = tt_ids.reshape(nblk, ROWS_BLK, 1)
    out = pl.pallas_call(
        _tc_epilogue,
        grid=(nblk,),
        in_specs=[
            pl.BlockSpec((ROWS_BLK, D), lambda i: (i, 0)),
            pl.BlockSpec((ROWS_BLK, D), lambda i: (i % pos_blks, 0)),
            pl.BlockSpec((2, D), lambda i: (0, 0)),
            pl.BlockSpec((1, ROWS_BLK, 1), lambda i: (i, 0, 0)),
            pl.BlockSpec((1, D), lambda i: (0, 0)),
            pl.BlockSpec((1, D), lambda i: (0, 0)),
        ],
        out_specs=pl.BlockSpec((ROWS_BLK, D), lambda i: (i, 0)),
        out_shape=jax.ShapeDtypeStruct((N, D), jnp.float32),
    )(gathered, position_embeddings, token_type_embeddings, tt3,
      ln_gamma.reshape(1, D), ln_beta.reshape(1, D))
    return out


def kernel(input_ids, token_type_ids, word_embeddings, position_embeddings,
           token_type_embeddings, ln_gamma, ln_beta):
    ids = input_ids.astype(jnp.int32).reshape(N)
    tt_ids = token_type_ids.astype(jnp.int32).reshape(N)
    gathered = _gather_rows(word_embeddings, ids)
    out = _epilogue(gathered, position_embeddings, token_type_embeddings,
                    tt_ids, ln_gamma, ln_beta)
    return out.reshape(B, S, D)


# speedup vs baseline: 2.0352x; 1.0430x over previous
import functools

import jax
import jax.numpy as jnp
from jax import lax
from jax.experimental import pallas as pl
from jax.experimental.pallas import tpu as pltpu
from jax.experimental.pallas import tpu_sc as plsc

B, S, D = 4, 2048, 768
N = B * S
EPS = 1e-12

_info = plsc.get_sparse_core_info()
NC, NS = _info.num_cores, _info.num_subcores
NW = NC * NS
TOK_PER_W = N // NW
CHUNK = 64
NCHUNK = TOK_PER_W // CHUNK


def _sc_gather(word_hbm, ids_hbm, out_hbm, idx_v, buf0, buf1, gs0, gs1, ss0,
               ss1):
    wid = lax.axis_index("s") * NC + lax.axis_index("c")
    pltpu.sync_copy(ids_hbm.at[wid], idx_v)
    base = wid * TOK_PER_W
    bufs = (buf0, buf1)
    gsems = (gs0, gs1)
    ssems = (ss0, ss1)
    gathers = [None] * NCHUNK
    scatters = [None] * NCHUNK
    gathers[0] = pltpu.async_copy(word_hbm.at[idx_v.at[0]], bufs[0], gsems[0])
    for j in range(NCHUNK):
        b = j % 2
        if j + 1 < NCHUNK:
            if j - 1 >= 0:
                scatters[j - 1].wait()
            gathers[j + 1] = pltpu.async_copy(
                word_hbm.at[idx_v.at[j + 1]], bufs[1 - b], gsems[1 - b])
        gathers[j].wait()
        scatters[j] = pltpu.async_copy(
            bufs[b], out_hbm.at[pl.ds(base + j * CHUNK, CHUNK)], ssems[b])
    scatters[NCHUNK - 2].wait()
    scatters[NCHUNK - 1].wait()


@jax.jit
def _gather_rows(word_embeddings, ids):
    ids3 = ids.reshape(NW, NCHUNK, CHUNK)
    mesh = plsc.VectorSubcoreMesh(core_axis_name="c", subcore_axis_name="s")
    return pl.kernel(
        _sc_gather,
        mesh=mesh,
        out_type=jax.ShapeDtypeStruct((N, D), jnp.float32),
        scratch_types=[
            pltpu.VMEM((NCHUNK, CHUNK), jnp.int32),
            pltpu.VMEM((CHUNK, D), jnp.float32),
            pltpu.VMEM((CHUNK, D), jnp.float32),
            pltpu.SemaphoreType.DMA,
            pltpu.SemaphoreType.DMA,
            pltpu.SemaphoreType.DMA,
            pltpu.SemaphoreType.DMA,
        ],
    )(word_embeddings, ids3)


ROWS_BLK = 256


def _tc_epilogue(g_ref, p_ref, tt_tab_ref, tt_ref, gamma_ref, beta_ref, o_ref):
    x = g_ref[...] + p_ref[...]
    tt = tt_ref[0].astype(jnp.float32)
    row0 = tt_tab_ref[0:1, :]
    row1 = tt_tab_ref[1:2, :]
    x = x + row0 + tt * (row1 - row0)
    mean = jnp.mean(x, axis=-1, keepdims=True)
    d = x - mean
    var = jnp.mean(d * d, axis=-1, keepdims=True)
    o_ref[...] = d * lax.rsqrt(var + EPS) * gamma_ref[...] + beta_ref[...]


@jax.jit
def _epilogue(gathered, position_embeddings, token_type_embeddings, tt_ids,
              ln_gamma, ln_beta):
    nblk = N // ROWS_BLK
    pos_blks = S // ROWS_BLK
    tt3 = tt_ids.reshape(nblk, ROWS_BLK, 1)
    out = pl.pallas_call(
        _tc_epilogue,
        grid=(pos_blks, B),
        in_specs=[
            pl.BlockSpec((ROWS_BLK, D), lambda p, b: (b * pos_blks + p, 0)),
            pl.BlockSpec((ROWS_BLK, D), lambda p, b: (p, 0)),
            pl.BlockSpec((2, D), lambda p, b: (0, 0)),
            pl.BlockSpec((1, ROWS_BLK, 1), lambda p, b: (b * pos_blks + p, 0, 0)),
            pl.BlockSpec((1, D), lambda p, b: (0, 0)),
            pl.BlockSpec((1, D), lambda p, b: (0, 0)),
        ],
        out_specs=pl.BlockSpec((ROWS_BLK, D), lambda p, b: (b * pos_blks + p, 0)),
        out_shape=jax.ShapeDtypeStruct((N, D), jnp.float32),
    )(gathered, position_embeddings, token_type_embeddings, tt3,
      ln_gamma.reshape(1, D), ln_beta.reshape(1, D))
    return out


def kernel(input_ids, token_type_ids, word_embeddings, position_embeddings,
           token_type_embeddings, ln_gamma, ln_beta):
    ids = input_ids.astype(jnp.int32).reshape(N)
    tt_ids = token_type_ids.astype(jnp.int32).reshape(N)
    gathered = _gather_rows(word_embeddings, ids)
    out = _epilogue(gathered, position_embeddings, token_type_embeddings,
                    tt_ids, ln_gamma, ln_beta)
    return out.reshape(B, S, D)
